# baseline (speedup 1.0000x reference)
import functools

import jax
import jax.numpy as jnp
from jax import lax
from jax.experimental import pallas as pl
from jax.experimental.pallas import tpu as pltpu
from jax.experimental.pallas import tpu_sc as plsc

USER_NUM_ = 1000000
HALF_U = USER_NUM_ // 2
ITEM_NUM_ = 100000
DIM_ = 64
BATCH_ = 16384

NC = 2
NS = 16
NW = NC * NS
B_PER_W = BATCH_ // NW
CHUNK = 128
N_CHUNKS = B_PER_W // CHUNK
GROUPS_PER_CHUNK = CHUNK // 16


def _gmf_body(ufull_hbm, utop_hbm, ubot_hbm, items_hbm,
              ut_top_hbm, ut_bot_hbm, it_hbm, w_hbm, b_hbm, out_hbm,
              uidx, uidxt, uidxb, iidx,
              ubT0, ubB0, ib0, ubT1, ubB1, ib1,
              wv, bv, outv, sem0, sem1):
    c = lax.axis_index("c")
    s = lax.axis_index("s")
    wid = s * NC + c
    base = wid * B_PER_W

    for j in range(N_CHUNKS):
        sl = pl.ds(base + j * CHUNK, CHUNK)
        pltpu.sync_copy(ufull_hbm.at[sl], uidx.at[j])
        pltpu.sync_copy(utop_hbm.at[sl], uidxt.at[j])
        pltpu.sync_copy(ubot_hbm.at[sl], uidxb.at[j])
        pltpu.sync_copy(items_hbm.at[sl], iidx.at[j])
    pltpu.sync_copy(w_hbm, wv)
    pltpu.sync_copy(b_hbm, bv)

    bufs = ((ubT0, ubB0, ib0), (ubT1, ubB1, ib1))
    sems = (sem0, sem1)

    def fire(j):
        sl = j % 2
        bt, bb, bi = bufs[sl]
        return (pltpu.async_copy(ut_top_hbm.at[uidxt.at[j]], bt, sems[sl]),
                pltpu.async_copy(ut_bot_hbm.at[uidxb.at[j]], bb, sems[sl]),
                pltpu.async_copy(it_hbm.at[iidx.at[j]], bi, sems[sl]))

    lane = lax.iota(jnp.int32, 16)
    bias = bv[:]
    wchunks = [wv[pl.ds(k * 16, 16)] for k in range(DIM_ // 16)]

    inflight = fire(0)
    for j in range(N_CHUNKS):
        nxt = fire(j + 1) if j + 1 < N_CHUNKS else None
        for cp in inflight:
            cp.wait()
        inflight = nxt
        bt, bb, bi = bufs[j % 2]

        def gbody(g, _):
            rows = g * 16 + lane
            ufull = uidx[j, pl.ds(g * 16, 16)]
            msk = ufull < HALF_U
            accs = [bias,
                    jnp.zeros((16,), jnp.float32),
                    jnp.zeros((16,), jnp.float32),
                    jnp.zeros((16,), jnp.float32)]
            for d in range(DIM_):
                col = jnp.full((16,), d, jnp.int32)
                ut = plsc.load_gather(bt, [rows, col])
                ub = plsc.load_gather(bb, [rows, col])
                uu = jnp.where(msk, ut, ub)
                vv = plsc.load_gather(bi, [rows, col])
                wd = wchunks[d // 16][d % 16]
                accs[d % 4] = accs[d % 4] + uu * vv * wd
            acc = (accs[0] + accs[1]) + (accs[2] + accs[3])
            outv[pl.ds(j * CHUNK + g * 16, 16)] = acc
            return 0

        lax.fori_loop(0, GROUPS_PER_CHUNK, gbody, 0)

    pltpu.sync_copy(outv, out_hbm.at[pl.ds(base, B_PER_W)])


@jax.jit
def _gmf_call(ufull, items, user_table, item_table, w_flat, bias_vec):
    utop = jnp.minimum(ufull, HALF_U - 1)
    ubot = jnp.maximum(ufull - HALF_U, 0)
    ut_top = lax.slice(user_table, (0, 0), (HALF_U, DIM_))
    ut_bot = lax.slice(user_table, (HALF_U, 0), (USER_NUM_, DIM_))
    mesh = plsc.VectorSubcoreMesh(core_axis_name="c", subcore_axis_name="s")
    buf = pltpu.VMEM((CHUNK, DIM_), jnp.float32)
    return pl.kernel(
        _gmf_body,
        mesh=mesh,
        compiler_params=pltpu.CompilerParams(
            needs_layout_passes=False, use_tc_tiling_on_sc=False),
        out_type=jax.ShapeDtypeStruct((BATCH_,), jnp.float32),
        scratch_types=[
            pltpu.VMEM((N_CHUNKS, CHUNK), jnp.int32),
            pltpu.VMEM((N_CHUNKS, CHUNK), jnp.int32),
            pltpu.VMEM((N_CHUNKS, CHUNK), jnp.int32),
            pltpu.VMEM((N_CHUNKS, CHUNK), jnp.int32),
            buf, buf, buf,
            buf, buf, buf,
            pltpu.VMEM((DIM_,), jnp.float32),
            pltpu.VMEM((16,), jnp.float32),
            pltpu.VMEM((B_PER_W,), jnp.float32),
            pltpu.SemaphoreType.DMA,
            pltpu.SemaphoreType.DMA,
        ],
    )(ufull, utop, ubot, items, ut_top, ut_bot, item_table,
      w_flat, bias_vec)


def kernel(users, items, user_table, item_table, beta_w, beta_b):
    users_i = users.astype(jnp.int32)
    items_i = items.astype(jnp.int32)
    w_flat = beta_w.reshape(DIM_)
    bias_vec = jnp.broadcast_to(beta_b, (16,))
    out = _gmf_call(users_i, items_i, user_table, item_table,
                    w_flat, bias_vec)
    return out.reshape(BATCH_, 1)


# speedup vs baseline: 2.7253x; 2.7253x over previous
import functools

import jax
import jax.numpy as jnp
from jax import lax
from jax.experimental import pallas as pl
from jax.experimental.pallas import tpu as pltpu
from jax.experimental.pallas import tpu_sc as plsc

USER_NUM_ = 1000000
ITEM_NUM_ = 100000
DIM_ = 64
BATCH_ = 16384

NC = 2
NS = 16
NW = NC * NS
B_PER_W = BATCH_ // NW
N_GROUPS = B_PER_W // 16


def _gmf_body(users_hbm, items_hbm, ut_hbm, it_hbm, w_hbm, b_hbm, out_hbm,
              uidx, iidx, urows, irows, wv, bv, outv, u_s, i_s,
              sem0, sem1, sem2, sem3):
    sems = (sem0, sem1, sem2, sem3)
    c = lax.axis_index("c")
    s = lax.axis_index("s")
    wid = s * NC + c
    base = wid * B_PER_W

    pltpu.sync_copy(users_hbm.at[pl.ds(base, B_PER_W)], uidx)
    pltpu.sync_copy(items_hbm.at[pl.ds(base, B_PER_W)], iidx)
    pltpu.sync_copy(w_hbm, wv)
    pltpu.sync_copy(b_hbm, bv)

    def mirror(g, _):
        uvec = uidx[pl.ds(g * 16, 16)]
        ivec = iidx[pl.ds(g * 16, 16)]
        for i in range(16):
            u_s[g * 16 + i] = uvec[i]
            i_s[g * 16 + i] = ivec[i]
        return 0

    lax.fori_loop(0, N_GROUPS, mirror, 0)

    lane = lax.iota(jnp.int32, 16)
    bias = bv[:]
    HALF = B_PER_W // 2

    for p in range(2):
        off = p * HALF

        def fire(q, _):
            for i in range(16):
                j = q * 16 + i
                pltpu.async_copy(ut_hbm.at[pl.ds(u_s[off + j], 1)],
                                 urows.at[pl.ds(j, 1)], sems[i % 4])
                pltpu.async_copy(it_hbm.at[pl.ds(i_s[off + j], 1)],
                                 irows.at[pl.ds(j, 1)], sems[i % 4])
            return 0

        lax.fori_loop(0, HALF // 16, fire, 0)

        for k in range(4):
            pltpu.make_async_copy(ut_hbm.at[pl.ds(0, 128)],
                                  urows.at[pl.ds(0, 128)], sems[k]).wait()

        def gbody(g, _):
            rows = g * 16 + lane
            wchunks = [wv[pl.ds(k * 16, 16)] for k in range(DIM_ // 16)]
            accs = [bias,
                    jnp.zeros((16,), jnp.float32),
                    jnp.zeros((16,), jnp.float32),
                    jnp.zeros((16,), jnp.float32)]
            for d in range(DIM_):
                col = jnp.full((16,), d, jnp.int32)
                uu = plsc.load_gather(urows, [rows, col])
                vv = plsc.load_gather(irows, [rows, col])
                wd = wchunks[d // 16][d % 16]
                accs[d % 4] = accs[d % 4] + uu * vv * wd
            acc = (accs[0] + accs[1]) + (accs[2] + accs[3])
            outv[pl.ds(off + g * 16, 16)] = acc
            return 0

        lax.fori_loop(0, HALF // 16, gbody, 0)

    pltpu.sync_copy(outv, out_hbm.at[pl.ds(base, B_PER_W)])


@jax.jit
def _gmf_call(users, items, user_table, item_table, w_flat, bias_vec):
    mesh = plsc.VectorSubcoreMesh(core_axis_name="c", subcore_axis_name="s")
    return pl.kernel(
        _gmf_body,
        mesh=mesh,
        compiler_params=pltpu.CompilerParams(needs_layout_passes=False),
        out_type=jax.ShapeDtypeStruct((BATCH_,), jnp.float32),
        scratch_types=[
            pltpu.VMEM((B_PER_W,), jnp.int32),
            pltpu.VMEM((B_PER_W,), jnp.int32),
            pltpu.VMEM((B_PER_W // 2, DIM_), jnp.float32),
            pltpu.VMEM((B_PER_W // 2, DIM_), jnp.float32),
            pltpu.VMEM((DIM_,), jnp.float32),
            pltpu.VMEM((16,), jnp.float32),
            pltpu.VMEM((B_PER_W,), jnp.float32),
            pltpu.SMEM((B_PER_W,), jnp.int32),
            pltpu.SMEM((B_PER_W,), jnp.int32),
            pltpu.SemaphoreType.DMA,
            pltpu.SemaphoreType.DMA,
            pltpu.SemaphoreType.DMA,
            pltpu.SemaphoreType.DMA,
        ],
    )(users, items, user_table, item_table, w_flat, bias_vec)


def kernel(users, items, user_table, item_table, beta_w, beta_b):
    users_i = users.astype(jnp.int32)
    items_i = items.astype(jnp.int32)
    w_flat = beta_w.reshape(DIM_)
    bias_vec = jnp.broadcast_to(beta_b, (16,))
    out = _gmf_call(users_i, items_i, user_table, item_table, w_flat, bias_vec)
    return out.reshape(BATCH_, 1)


# speedup vs baseline: 2.7380x; 1.0046x over previous
import functools

import jax
import jax.numpy as jnp
from jax import lax
from jax.experimental import pallas as pl
from jax.experimental.pallas import tpu as pltpu
from jax.experimental.pallas import tpu_sc as plsc

USER_NUM_ = 1000000
ITEM_NUM_ = 100000
DIM_ = 64
BATCH_ = 16384

NC = 2
NS = 16
NW = NC * NS
B_PER_W = BATCH_ // NW
N_GROUPS = B_PER_W // 16


def _gmf_body(users_hbm, items_hbm, ut_hbm, it_hbm, w_hbm, b_hbm, out_hbm,
              uidx, iidx, urows, irows, wv, bv, outv, u_s, i_s, sem):
    c = lax.axis_index("c")
    s = lax.axis_index("s")
    wid = s * NC + c
    base = wid * B_PER_W

    pltpu.sync_copy(users_hbm.at[pl.ds(base, B_PER_W)], uidx)
    pltpu.sync_copy(items_hbm.at[pl.ds(base, B_PER_W)], iidx)
    pltpu.sync_copy(w_hbm, wv)
    pltpu.sync_copy(b_hbm, bv)

    def mirror(g, _):
        uvec = uidx[pl.ds(g * 16, 16)]
        ivec = iidx[pl.ds(g * 16, 16)]
        for i in range(16):
            u_s[g * 16 + i] = uvec[i]
            i_s[g * 16 + i] = ivec[i]
        return 0

    lax.fori_loop(0, N_GROUPS, mirror, 0)

    lane = lax.iota(jnp.int32, 16)
    bias = bv[:]
    HALF = B_PER_W // 2

    for p in range(2):
        off = p * HALF

        def fire(q, _):
            for i in range(16):
                j = q * 16 + i
                pltpu.async_copy(ut_hbm.at[pl.ds(u_s[off + j], 1)],
                                 urows.at[pl.ds(j, 1)], sem)
                pltpu.async_copy(it_hbm.at[pl.ds(i_s[off + j], 1)],
                                 irows.at[pl.ds(j, 1)], sem)
            return 0

        lax.fori_loop(0, HALF // 16, fire, 0)

        def gbody(g, _):
            pltpu.make_async_copy(ut_hbm.at[pl.ds(0, 16)],
                                  urows.at[pl.ds(0, 16)], sem).wait()
            pltpu.make_async_copy(it_hbm.at[pl.ds(0, 16)],
                                  irows.at[pl.ds(0, 16)], sem).wait()
            rows = g * 16 + lane
            wchunks = [wv[pl.ds(k * 16, 16)] for k in range(DIM_ // 16)]
            accs = [bias,
                    jnp.zeros((16,), jnp.float32),
                    jnp.zeros((16,), jnp.float32),
                    jnp.zeros((16,), jnp.float32)]
            for d in range(DIM_):
                col = jnp.full((16,), d, jnp.int32)
                uu = plsc.load_gather(urows, [rows, col])
                vv = plsc.load_gather(irows, [rows, col])
                wd = wchunks[d // 16][d % 16]
                accs[d % 4] = accs[d % 4] + uu * vv * wd
            acc = (accs[0] + accs[1]) + (accs[2] + accs[3])
            outv[pl.ds(off + g * 16, 16)] = acc
            return 0

        lax.fori_loop(0, HALF // 16, gbody, 0)

    pltpu.sync_copy(outv, out_hbm.at[pl.ds(base, B_PER_W)])


@jax.jit
def _gmf_call(users, items, user_table, item_table, w_flat, bias_vec):
    mesh = plsc.VectorSubcoreMesh(core_axis_name="c", subcore_axis_name="s")
    return pl.kernel(
        _gmf_body,
        mesh=mesh,
        compiler_params=pltpu.CompilerParams(needs_layout_passes=False),
        out_type=jax.ShapeDtypeStruct((BATCH_,), jnp.float32),
        scratch_types=[
            pltpu.VMEM((B_PER_W,), jnp.int32),
            pltpu.VMEM((B_PER_W,), jnp.int32),
            pltpu.VMEM((B_PER_W // 2, DIM_), jnp.float32),
            pltpu.VMEM((B_PER_W // 2, DIM_), jnp.float32),
            pltpu.VMEM((DIM_,), jnp.float32),
            pltpu.VMEM((16,), jnp.float32),
            pltpu.VMEM((B_PER_W,), jnp.float32),
            pltpu.SMEM((B_PER_W,), jnp.int32),
            pltpu.SMEM((B_PER_W,), jnp.int32),
            pltpu.SemaphoreType.DMA,
        ],
    )(users, items, user_table, item_table, w_flat, bias_vec)


def kernel(users, items, user_table, item_table, beta_w, beta_b):
    users_i = users.astype(jnp.int32)
    items_i = items.astype(jnp.int32)
    w_flat = beta_w.reshape(DIM_)
    bias_vec = jnp.broadcast_to(beta_b, (16,))
    out = _gmf_call(users_i, items_i, user_table, item_table, w_flat, bias_vec)
    return out.reshape(BATCH_, 1)
